# baseline (speedup 1.0000x reference)
import jax
import jax.numpy as jnp
from jax.experimental import pallas as pl
from jax.experimental.pallas import tpu as pltpu

_EPS = 1e-12


def _netvlad_block(x_ref, w_ref, c_ref, o_ref):
    x = x_ref[0]
    w = w_ref[...]
    c = c_ref[...]

    nrm = jnp.sqrt(jnp.sum(x * x, axis=0, keepdims=True))
    xn = x / jnp.maximum(nrm, _EPS)

    logits = jnp.dot(w, xn, preferred_element_type=jnp.float32)
    m = jnp.max(logits, axis=0, keepdims=True)
    e = jnp.exp(logits - m)
    a = e / jnp.sum(e, axis=0, keepdims=True)

    vlad = jax.lax.dot_general(
        a, xn, (((1,), (1,)), ((), ())), preferred_element_type=jnp.float32
    )
    vlad = vlad - jnp.sum(a, axis=1, keepdims=True) * c

    rn = jnp.sqrt(jnp.sum(vlad * vlad, axis=1, keepdims=True))
    vlad = vlad / jnp.maximum(rn, _EPS)
    g = jnp.sqrt(jnp.sum(vlad * vlad, keepdims=True))
    o_ref[0] = vlad / jnp.maximum(g, _EPS)


@jax.jit
def kernel(x, conv_w, centroids):
    B, D, H, W = x.shape
    K = centroids.shape[0]
    N = H * W
    xf = x.reshape(B, D, N)
    out = pl.pallas_call(
        _netvlad_block,
        grid=(B,),
        in_specs=[
            pl.BlockSpec((1, D, N), lambda b: (b, 0, 0)),
            pl.BlockSpec((K, D), lambda b: (0, 0)),
            pl.BlockSpec((K, D), lambda b: (0, 0)),
        ],
        out_specs=pl.BlockSpec((1, K, D), lambda b: (b, 0, 0)),
        out_shape=jax.ShapeDtypeStruct((B, K, D), jnp.float32),
        compiler_params=pltpu.CompilerParams(
            dimension_semantics=("parallel",),
        ),
    )(xf, conv_w, centroids)
    return out.reshape(B, K * D)


# speedup vs baseline: 1.9360x; 1.9360x over previous
import jax
import jax.numpy as jnp
from jax.experimental import pallas as pl
from jax.experimental.pallas import tpu as pltpu

_EPS = 1e-12


def _netvlad_block(x_ref, wt_ref, c_ref, o_ref):
    x = x_ref[0]
    wt = wt_ref[...]
    c = c_ref[...]

    ssq = jnp.sum(x * x, axis=1, keepdims=True)
    xn = x / jnp.maximum(jnp.sqrt(ssq), _EPS)

    logits = jnp.dot(xn, wt, preferred_element_type=jnp.float32)
    m = jnp.max(logits, axis=1, keepdims=True)
    e = jnp.exp(logits - m)
    a = e / jnp.sum(e, axis=1, keepdims=True)

    vlad = jax.lax.dot_general(
        a, xn, (((0,), (0,)), ((), ())), preferred_element_type=jnp.float32
    )
    asum = jnp.sum(a, axis=0, keepdims=True)
    vlad = vlad - asum.T * c

    rn = jnp.sqrt(jnp.sum(vlad * vlad, axis=1, keepdims=True))
    vlad = vlad / jnp.maximum(rn, _EPS)
    g = jnp.sqrt(jnp.sum(vlad * vlad, keepdims=True))
    o_ref[0] = vlad / jnp.maximum(g, _EPS)


@jax.jit
def kernel(x, conv_w, centroids):
    B, D, H, W = x.shape
    K = centroids.shape[0]
    N = H * W
    xt = jnp.transpose(x, (0, 2, 3, 1)).reshape(B, N, D)
    out = pl.pallas_call(
        _netvlad_block,
        grid=(B,),
        in_specs=[
            pl.BlockSpec((1, N, D), lambda b: (b, 0, 0)),
            pl.BlockSpec((D, K), lambda b: (0, 0)),
            pl.BlockSpec((K, D), lambda b: (0, 0)),
        ],
        out_specs=pl.BlockSpec((1, K, D), lambda b: (b, 0, 0)),
        out_shape=jax.ShapeDtypeStruct((B, K, D), jnp.float32),
        compiler_params=pltpu.CompilerParams(
            dimension_semantics=("parallel",),
        ),
    )(xt, conv_w.T, centroids)
    return out.reshape(B, K * D)


# speedup vs baseline: 1.9430x; 1.0036x over previous
import jax
import jax.numpy as jnp
from jax.experimental import pallas as pl
from jax.experimental.pallas import tpu as pltpu

_EPS = 1e-12


def _netvlad_block(x_ref, wt_ref, c_ref, o_ref):
    x = x_ref[0]
    wt = wt_ref[...]
    c = c_ref[...]

    ssq = jnp.sum(x * x, axis=1, keepdims=True)
    rinv = jax.lax.rsqrt(jnp.maximum(ssq, _EPS * _EPS))

    logits = jnp.dot(x, wt, preferred_element_type=jnp.float32) * rinv
    m = jnp.max(logits, axis=1, keepdims=True)
    e = jnp.exp(logits - m)
    a = e / jnp.sum(e, axis=1, keepdims=True)

    a2 = a * rinv
    vlad = jax.lax.dot_general(
        a2, x, (((0,), (0,)), ((), ())), preferred_element_type=jnp.float32
    )
    asum = jnp.sum(a, axis=0, keepdims=True)
    vlad = vlad - asum.T * c

    rn = jnp.sqrt(jnp.sum(vlad * vlad, axis=1, keepdims=True))
    vlad = vlad / jnp.maximum(rn, _EPS)
    g = jnp.sqrt(jnp.sum(vlad * vlad, keepdims=True))
    o_ref[0] = vlad / jnp.maximum(g, _EPS)


@jax.jit
def kernel(x, conv_w, centroids):
    B, D, H, W = x.shape
    K = centroids.shape[0]
    N = H * W
    xt = jnp.transpose(x, (0, 2, 3, 1)).reshape(B, N, D)
    out = pl.pallas_call(
        _netvlad_block,
        grid=(B,),
        in_specs=[
            pl.BlockSpec((1, N, D), lambda b: (b, 0, 0)),
            pl.BlockSpec((D, K), lambda b: (0, 0)),
            pl.BlockSpec((K, D), lambda b: (0, 0)),
        ],
        out_specs=pl.BlockSpec((1, K, D), lambda b: (b, 0, 0)),
        out_shape=jax.ShapeDtypeStruct((B, K, D), jnp.float32),
        compiler_params=pltpu.CompilerParams(
            dimension_semantics=("parallel",),
        ),
    )(xt, conv_w.T, centroids)
    return out.reshape(B, K * D)


# speedup vs baseline: 2.0125x; 1.0357x over previous
import jax
import jax.numpy as jnp
from jax.experimental import pallas as pl
from jax.experimental.pallas import tpu as pltpu

_EPS = 1e-12


def _half_vlad(x, wt):
    ssq = jnp.sum(x * x, axis=1, keepdims=True)
    rinv = jax.lax.rsqrt(jnp.maximum(ssq, _EPS * _EPS))
    logits = jnp.dot(x, wt, preferred_element_type=jnp.float32) * rinv
    m = jnp.max(logits, axis=1, keepdims=True)
    e = jnp.exp(logits - m)
    a = e / jnp.sum(e, axis=1, keepdims=True)
    a2 = a * rinv
    vlad = jax.lax.dot_general(
        a2, x, (((0,), (0,)), ((), ())), preferred_element_type=jnp.float32
    )
    asum = jnp.sum(a, axis=0, keepdims=True)
    return vlad, asum


def _netvlad_block(x1_ref, x2_ref, wt_ref, c_ref, o_ref):
    wt = wt_ref[...]
    c = c_ref[...]

    v1, s1 = _half_vlad(x1_ref[0, 0], wt)
    v2, s2 = _half_vlad(x2_ref[0, 0], wt)
    vlad = (v1 + v2) - (s1 + s2).T * c

    rn = jnp.sqrt(jnp.sum(vlad * vlad, axis=1, keepdims=True))
    vlad = vlad / jnp.maximum(rn, _EPS)
    g = jnp.sqrt(jnp.sum(vlad * vlad, keepdims=True))
    o_ref[0] = vlad / jnp.maximum(g, _EPS)


@jax.jit
def kernel(x, conv_w, centroids):
    B, D, H, W = x.shape
    K = centroids.shape[0]
    N = H * W
    Nh = N // 2
    xt = jnp.transpose(x, (0, 2, 3, 1)).reshape(B, 2, Nh, D)
    out = pl.pallas_call(
        _netvlad_block,
        grid=(B,),
        in_specs=[
            pl.BlockSpec((1, 1, Nh, D), lambda b: (b, 0, 0, 0)),
            pl.BlockSpec((1, 1, Nh, D), lambda b: (b, 1, 0, 0)),
            pl.BlockSpec((D, K), lambda b: (0, 0)),
            pl.BlockSpec((K, D), lambda b: (0, 0)),
        ],
        out_specs=pl.BlockSpec((1, K, D), lambda b: (b, 0, 0)),
        out_shape=jax.ShapeDtypeStruct((B, K, D), jnp.float32),
        compiler_params=pltpu.CompilerParams(
            dimension_semantics=("parallel",),
        ),
    )(xt, xt, conv_w.T, centroids)
    return out.reshape(B, K * D)
